# baseline (speedup 1.0000x reference)
import functools

import jax
import jax.numpy as jnp
import numpy as np
from jax import lax
from jax.experimental import pallas as pl
from jax.experimental.pallas import tpu as pltpu
from jax.experimental.pallas import tpu_sc as plsc

_VOCAB = 100000
_D = 1024
_B = 4
_S = 4096
_SCALE = np.float32(1.0 / np.sqrt(_D))

_NC = 2
_NS = 16
_NW = _NC * _NS
_L = 16

_N = _B * _S
_SPW = _S // _NW
_R = 8
_NCH = _SPW // _R
_PW = _B * _SPW
_NBANK = 3


def _pos_encoding() -> np.ndarray:
    pos = np.arange(_S, dtype=np.float32)[:, None]
    div = np.exp(
        np.arange(0, _D, 2, dtype=np.float32) * (-np.log(10000.0) / _D)
    )
    pe = np.zeros((_S, _D), dtype=np.float32)
    pe[:, 0::2] = np.sin(pos * div)
    pe[:, 1::2] = np.cos(pos * div)
    return pe


_PE = jax.numpy.asarray(_pos_encoding().reshape(-1))


def _sc_body(idx_hbm, pe_hbm, table_hbm, out_hbm,
             idx_v, x0, x1, x2, p0, p1, p2, g0, g1, g2, s0_, s1_, s2_):
    cid = lax.axis_index("c")
    sid = lax.axis_index("s")
    wid = sid * _NC + cid
    s0 = wid * _SPW

    xb = (x0, x1, x2)
    pb = (p0, p1, p2)
    gs = (g0, g1, g2)
    ss = (s0_, s1_, s2_)

    for b in range(_B):
        pltpu.sync_copy(idx_hbm.at[pl.ds(b * _S + s0, _SPW)],
                        idx_v.at[pl.ds(b * _SPW, _SPW)])

    def fire_gather(c, k):
        pltpu.async_copy(pe_hbm.at[pl.ds((s0 + c * _R) * _D, _R * _D)],
                         pb[k], gs[k])
        for b in range(_B):
            pltpu.async_copy(
                table_hbm.at[idx_v.at[pl.ds(b * _SPW + c * _R, _R)]],
                xb[k].at[b], gs[k])

    def wait_gather(c, k):
        pltpu.make_async_copy(pe_hbm.at[pl.ds((s0 + c * _R) * _D, _R * _D)],
                              pb[k], gs[k]).wait()
        for b in range(_B):
            pltpu.make_async_copy(
                table_hbm.at[idx_v.at[pl.ds(b * _SPW + c * _R, _R)]],
                xb[k].at[b], gs[k]).wait()

    def fire_scatter(c, k):
        pltpu.async_copy(xb[k].at[pl.ds(0, _B)],
                         out_hbm.at[:, pl.ds(s0 + c * _R, _R), :], ss[k])

    def wait_scatter(c, k):
        pltpu.make_async_copy(xb[k].at[pl.ds(0, _B)],
                              out_hbm.at[:, pl.ds(s0 + c * _R, _R), :],
                              ss[k]).wait()

    def fma(k):
        xk = xb[k]
        pk = pb[k]

        def row_body(r, carry):
            for col in range(_D // _L):
                sl = pl.ds(col * _L, _L)
                p = pk[pl.ds(r * _D + col * _L, _L)]
                for b in range(_B):
                    xk[b, r, sl] = xk[b, r, sl] * _SCALE + p
            return carry

        lax.fori_loop(0, _R, row_body, 0)

    def step(c, k, first, last_fire):
        wait_gather(c, k)
        fma(k)
        fire_scatter(c, k)
        kn = (k + 2) % _NBANK
        if first is None:
            wait_scatter(c - 1, kn)
        else:
            @pl.when(first > 0)
            def _():
                wait_scatter(c - 1, kn)
        if last_fire is None:
            fire_gather(c + 2, kn)
        else:
            @pl.when(last_fire)
            def _():
                fire_gather(c + 2, kn)

    fire_gather(0, 0)
    fire_gather(1, 1)

    def iter_body(t, carry):
        step(3 * t, 0, t, None)
        step(3 * t + 1, 1, None, None)
        step(3 * t + 2, 2, None, t < (_NCH // _NBANK - 1))
        return carry

    lax.fori_loop(0, _NCH // _NBANK, iter_body, 0)

    c = _NCH - 1
    wait_gather(c, 0)
    fma(0)
    fire_scatter(c, 0)
    wait_scatter(c - 1, 2)
    wait_scatter(c, 0)


@jax.jit
def _embed(idx_flat, table, pe):
    fn = functools.partial(
        pl.kernel,
        mesh=plsc.VectorSubcoreMesh(core_axis_name="c", subcore_axis_name="s"),
        out_type=jax.ShapeDtypeStruct((_B, _S, _D), jnp.float32),
        scratch_types=[
            pltpu.VMEM((_PW,), jnp.int32),
            pltpu.VMEM((_B, _R, _D), jnp.float32),
            pltpu.VMEM((_B, _R, _D), jnp.float32),
            pltpu.VMEM((_B, _R, _D), jnp.float32),
            pltpu.VMEM((_R * _D,), jnp.float32),
            pltpu.VMEM((_R * _D,), jnp.float32),
            pltpu.VMEM((_R * _D,), jnp.float32),
            pltpu.SemaphoreType.DMA,
            pltpu.SemaphoreType.DMA,
            pltpu.SemaphoreType.DMA,
            pltpu.SemaphoreType.DMA,
            pltpu.SemaphoreType.DMA,
            pltpu.SemaphoreType.DMA,
        ],
    )(_sc_body)
    return fn(idx_flat, pe, table)


def kernel(inputs, table):
    idx_flat = inputs.reshape(_N)
    return _embed(idx_flat, table, _PE)


# speedup vs baseline: 1.2156x; 1.2156x over previous
import functools

import jax
import jax.numpy as jnp
import numpy as np
from jax import lax
from jax.experimental import pallas as pl
from jax.experimental.pallas import tpu as pltpu
from jax.experimental.pallas import tpu_sc as plsc

_VOCAB = 100000
_D = 1024
_B = 4
_S = 4096
_SCALE = np.float32(1.0 / np.sqrt(_D))

_NC = 2
_NS = 16
_NW = _NC * _NS
_L = 16

_N = _B * _S
_SPW = _S // _NW
_R = 8
_NCH = _SPW // _R
_PW = _B * _SPW
_NBANK = 3


def _pos_encoding() -> np.ndarray:
    pos = np.arange(_S, dtype=np.float32)[:, None]
    div = np.exp(
        np.arange(0, _D, 2, dtype=np.float32) * (-np.log(10000.0) / _D)
    )
    pe = np.zeros((_S, _D), dtype=np.float32)
    pe[:, 0::2] = np.sin(pos * div)
    pe[:, 1::2] = np.cos(pos * div)
    return pe


def _pe_packed() -> np.ndarray:
    pe = _pos_encoding().reshape(_S, _D // 32, 2, _L)
    pb16 = pe.astype(jnp.bfloat16).view(np.uint16).astype(np.uint32)
    packed = pb16[:, :, 0, :] | (pb16[:, :, 1, :] << 16)
    return np.ascontiguousarray(packed.reshape(_S, _D // 2)).view(np.float32)


_PE_NP = _pe_packed()
_MASK_HI = np.int32(np.uint32(0xFFFF0000).astype(np.int64) - (1 << 32))


def _sc_body(idx_hbm, pe_hbm, table_hbm, out_hbm,
             idx_v, x0, x1, x2, g0, g1, g2, s0_, s1_, s2_):
    cid = lax.axis_index("c")
    sid = lax.axis_index("s")
    wid = sid * _NC + cid
    s0 = wid * _SPW

    xb = (x0, x1, x2)
    gs = (g0, g1, g2)
    ss = (s0_, s1_, s2_)

    for b in range(_B):
        pltpu.sync_copy(idx_hbm.at[pl.ds(b * _S + s0, _SPW)],
                        idx_v.at[pl.ds(b * _SPW, _SPW)])

    def fire_gather(c, k):
        pltpu.async_copy(pe_hbm.at[pl.ds(s0 + c * _R, _R)],
                         xb[k].at[_B, :, pl.ds(0, _D // 2)], gs[k])
        for b in range(_B):
            pltpu.async_copy(
                table_hbm.at[idx_v.at[pl.ds(b * _SPW + c * _R, _R)]],
                xb[k].at[b], gs[k])

    def wait_gather(c, k):
        pltpu.make_async_copy(pe_hbm.at[pl.ds(s0 + c * _R, _R)],
                              xb[k].at[_B, :, pl.ds(0, _D // 2)],
                              gs[k]).wait()
        for b in range(_B):
            pltpu.make_async_copy(
                table_hbm.at[idx_v.at[pl.ds(b * _SPW + c * _R, _R)]],
                xb[k].at[b], gs[k]).wait()

    def fire_scatter(c, k):
        pltpu.async_copy(xb[k].at[pl.ds(0, _B)],
                         out_hbm.at[:, pl.ds(s0 + c * _R, _R), :], ss[k])

    def wait_scatter(c, k):
        pltpu.make_async_copy(xb[k].at[pl.ds(0, _B)],
                              out_hbm.at[:, pl.ds(s0 + c * _R, _R), :],
                              ss[k]).wait()

    def fma(k):
        xk = xb[k]

        def row_body(r, carry):
            for g in range(_D // (2 * _L)):
                w = lax.bitcast_convert_type(
                    xk[_B, r, pl.ds(g * _L, _L)], jnp.int32)
                pa = lax.bitcast_convert_type(w << 16, jnp.float32)
                pc = lax.bitcast_convert_type(w & _MASK_HI, jnp.float32)
                for h, p in ((0, pa), (1, pc)):
                    sl = pl.ds(g * 2 * _L + h * _L, _L)
                    for b in range(_B):
                        xk[b, r, sl] = xk[b, r, sl] * _SCALE + p
            return carry

        lax.fori_loop(0, _R, row_body, 0)

    def step(c, k, first, last_fire):
        wait_gather(c, k)
        fma(k)
        fire_scatter(c, k)
        kn = (k + 2) % _NBANK
        if first is None:
            wait_scatter(c - 1, kn)
        else:
            @pl.when(first > 0)
            def _():
                wait_scatter(c - 1, kn)
        if last_fire is None:
            fire_gather(c + 2, kn)
        else:
            @pl.when(last_fire)
            def _():
                fire_gather(c + 2, kn)

    fire_gather(0, 0)
    fire_gather(1, 1)

    def iter_body(t, carry):
        step(3 * t, 0, t, None)
        step(3 * t + 1, 1, None, None)
        step(3 * t + 2, 2, None, t < (_NCH // _NBANK - 1))
        return carry

    lax.fori_loop(0, _NCH // _NBANK, iter_body, 0)

    c = _NCH - 1
    wait_gather(c, 0)
    fma(0)
    fire_scatter(c, 0)
    wait_scatter(c - 1, 2)
    wait_scatter(c, 0)


@jax.jit
def _embed(idx_flat, table, pe):
    fn = functools.partial(
        pl.kernel,
        mesh=plsc.VectorSubcoreMesh(core_axis_name="c", subcore_axis_name="s"),
        out_type=jax.ShapeDtypeStruct((_B, _S, _D), jnp.float32),
        scratch_types=[
            pltpu.VMEM((_PW,), jnp.int32),
            pltpu.VMEM((_B + 1, _R, _D), jnp.float32),
            pltpu.VMEM((_B + 1, _R, _D), jnp.float32),
            pltpu.VMEM((_B + 1, _R, _D), jnp.float32),
            pltpu.SemaphoreType.DMA,
            pltpu.SemaphoreType.DMA,
            pltpu.SemaphoreType.DMA,
            pltpu.SemaphoreType.DMA,
            pltpu.SemaphoreType.DMA,
            pltpu.SemaphoreType.DMA,
        ],
    )(_sc_body)
    return fn(idx_flat, pe, table)


def kernel(inputs, table):
    idx_flat = inputs.reshape(_N)
    return _embed(idx_flat, table, jnp.asarray(_PE_NP))


# speedup vs baseline: 1.2342x; 1.0153x over previous
import functools

import jax
import jax.numpy as jnp
import numpy as np
from jax import lax
from jax.experimental import pallas as pl
from jax.experimental.pallas import tpu as pltpu
from jax.experimental.pallas import tpu_sc as plsc

_VOCAB = 100000
_D = 1024
_B = 4
_S = 4096
_SCALE = np.float32(1.0 / np.sqrt(_D))

_NC = 2
_NS = 16
_NW = _NC * _NS
_L = 16

_N = _B * _S
_SPW = _S // _NW
_R = 8
_NCH = _SPW // _R
_PW = _B * _SPW
_NBANK = 3


def _pos_encoding() -> np.ndarray:
    pos = np.arange(_S, dtype=np.float32)[:, None]
    div = np.exp(
        np.arange(0, _D, 2, dtype=np.float32) * (-np.log(10000.0) / _D)
    )
    pe = np.zeros((_S, _D), dtype=np.float32)
    pe[:, 0::2] = np.sin(pos * div)
    pe[:, 1::2] = np.cos(pos * div)
    return pe


def _pe_packed() -> np.ndarray:
    pe = _pos_encoding().reshape(_S, _D // 32, 2, _L)
    pb16 = pe.astype(jnp.bfloat16).view(np.uint16).astype(np.uint32)
    packed = pb16[:, :, 0, :] | (pb16[:, :, 1, :] << 16)
    return np.ascontiguousarray(packed.reshape(_S, _D // 2)).view(np.float32)


_PE_NP = _pe_packed()
_MASK_HI = np.int32(np.uint32(0xFFFF0000).astype(np.int64) - (1 << 32))


def _sc_body(idx_hbm, pe_hbm, table_hbm, out_hbm,
             idx_v, x0, x1, x2, g0, g1, g2, s0_, s1_, s2_):
    cid = lax.axis_index("c")
    sid = lax.axis_index("s")
    wid = sid * _NC + cid
    s0 = wid * _SPW

    xb = (x0, x1, x2)
    gs = (g0, g1, g2)
    ss = (s0_, s1_, s2_)

    for b in range(_B):
        pltpu.sync_copy(idx_hbm.at[b, pl.ds(s0, _SPW)],
                        idx_v.at[pl.ds(b * _SPW, _SPW)])

    def fire_gather(c, k):
        pltpu.async_copy(pe_hbm.at[pl.ds(s0 + c * _R, _R)],
                         xb[k].at[_B, :, pl.ds(0, _D // 2)], gs[k])
        for b in range(_B):
            pltpu.async_copy(
                table_hbm.at[idx_v.at[pl.ds(b * _SPW + c * _R, _R)]],
                xb[k].at[b], gs[k])

    def wait_gather(c, k):
        pltpu.make_async_copy(pe_hbm.at[pl.ds(s0 + c * _R, _R)],
                              xb[k].at[_B, :, pl.ds(0, _D // 2)],
                              gs[k]).wait()
        for b in range(_B):
            pltpu.make_async_copy(
                table_hbm.at[idx_v.at[pl.ds(b * _SPW + c * _R, _R)]],
                xb[k].at[b], gs[k]).wait()

    def fire_scatter(c, k):
        pltpu.async_copy(xb[k].at[pl.ds(0, _B)],
                         out_hbm.at[:, pl.ds(s0 + c * _R, _R), :], ss[k])

    def wait_scatter(c, k):
        pltpu.make_async_copy(xb[k].at[pl.ds(0, _B)],
                              out_hbm.at[:, pl.ds(s0 + c * _R, _R), :],
                              ss[k]).wait()

    def fma(k):
        xk = xb[k]

        def row_body(r, carry):
            for g in range(_D // (2 * _L)):
                w = lax.bitcast_convert_type(
                    xk[_B, r, pl.ds(g * _L, _L)], jnp.int32)
                pa = lax.bitcast_convert_type(w << 16, jnp.float32)
                pc = lax.bitcast_convert_type(w & _MASK_HI, jnp.float32)
                for h, p in ((0, pa), (1, pc)):
                    sl = pl.ds(g * 2 * _L + h * _L, _L)
                    for b in range(_B):
                        xk[b, r, sl] = xk[b, r, sl] * _SCALE + p
            return carry

        lax.fori_loop(0, _R, row_body, 0)

    def step(c, k, first, last_fire):
        wait_gather(c, k)
        fma(k)
        fire_scatter(c, k)
        kn = (k + 2) % _NBANK
        if first is None:
            wait_scatter(c - 1, kn)
        else:
            @pl.when(first > 0)
            def _():
                wait_scatter(c - 1, kn)
        if last_fire is None:
            fire_gather(c + 2, kn)
        else:
            @pl.when(last_fire)
            def _():
                fire_gather(c + 2, kn)

    fire_gather(0, 0)
    fire_gather(1, 1)

    def iter_body(t, carry):
        step(3 * t, 0, t, None)
        step(3 * t + 1, 1, None, None)
        step(3 * t + 2, 2, None, t < (_NCH // _NBANK - 1))
        return carry

    lax.fori_loop(0, _NCH // _NBANK, iter_body, 0)

    c = _NCH - 1
    wait_gather(c, 0)
    fma(0)
    fire_scatter(c, 0)
    wait_scatter(c - 1, 2)
    wait_scatter(c, 0)


@jax.jit
def _embed(idx2d, table, pe):
    fn = functools.partial(
        pl.kernel,
        mesh=plsc.VectorSubcoreMesh(core_axis_name="c", subcore_axis_name="s"),
        out_type=jax.ShapeDtypeStruct((_B, _S, _D), jnp.float32),
        scratch_types=[
            pltpu.VMEM((_PW,), jnp.int32),
            pltpu.VMEM((_B + 1, _R, _D), jnp.float32),
            pltpu.VMEM((_B + 1, _R, _D), jnp.float32),
            pltpu.VMEM((_B + 1, _R, _D), jnp.float32),
            pltpu.SemaphoreType.DMA,
            pltpu.SemaphoreType.DMA,
            pltpu.SemaphoreType.DMA,
            pltpu.SemaphoreType.DMA,
            pltpu.SemaphoreType.DMA,
            pltpu.SemaphoreType.DMA,
        ],
    )(_sc_body)
    return fn(idx2d, pe, table)


def kernel(inputs, table):
    return _embed(inputs, table, jnp.asarray(_PE_NP))
